# baseline (speedup 1.0000x reference)
import functools

import jax
import jax.numpy as jnp
from jax import lax
from jax.experimental import pallas as pl
from jax.experimental.pallas import tpu as pltpu
from jax.experimental.pallas import tpu_sc as plsc

NN = 10000
NP = 10240
EE = 160000
TT = 8
DD = 256
HH = 256

NC = 2
NS = 16
BB = 32
NB = 320
EP = NS * NB * BB
ZR = NP // NS


@functools.lru_cache(maxsize=None)
def _sc_agg_multi(ntab):
    mesh = plsc.VectorSubcoreMesh(core_axis_name="c", subcore_axis_name="s",
                                  num_cores=NC, num_subcores=NS)
    nbuf = 8
    CH = 16
    NCHUNK = NB // CH

    @functools.partial(
        pl.kernel,
        out_type=[jax.ShapeDtypeStruct((2 * NP, 128), jnp.float32)] * ntab,
        mesh=mesh,
        scratch_types=[
            pltpu.VMEM((CH, BB), jnp.int32),
            pltpu.VMEM((CH, BB), jnp.int32),
            [pltpu.VMEM((BB, 128), jnp.float32)] * nbuf,
            pltpu.VMEM_SHARED((NP, 128), jnp.float32),
            [pltpu.SemaphoreType.DMA] * nbuf,
        ],
    )
    def sc_agg(*refs):
        tables = refs[:ntab]
        srcs_hbm, dsts_hbm, zeros_hbm = refs[ntab:ntab + 3]
        outs = refs[ntab + 3:2 * ntab + 3]
        src_v, dst_v, rows_v, acc, sems = refs[2 * ntab + 3:]
        c = lax.axis_index("c")
        s = lax.axis_index("s")

        def stage0_and_prime(table_hbm):
            pltpu.sync_copy(srcs_hbm.at[c, s, pl.ds(0, CH)], src_v)
            pltpu.sync_copy(dsts_hbm.at[s, pl.ds(0, CH)], dst_v)
            for b in range(nbuf):
                pltpu.async_copy(table_hbm.at[src_v.at[b]], rows_v[b],
                                 sems[b])

        def run_table(table_hbm):
            def chunk(k, carry):
                def one(j, refire):
                    b = j % nbuf
                    pltpu.make_async_copy(table_hbm.at[src_v.at[j]],
                                          rows_v[b], sems[b]).wait()
                    pltpu.sync_copy(rows_v[b], acc.at[dst_v.at[j]], add=True)
                    if refire:
                        @pl.when(j + nbuf < CH)
                        def _():
                            pltpu.async_copy(
                                table_hbm.at[src_v.at[j + nbuf]],
                                rows_v[b], sems[b])

                for j in range(0, CH - nbuf):
                    one(j, True)
                for j in range(CH - nbuf, CH):
                    one(j, False)

                @pl.when(k + 1 < NCHUNK)
                def _():
                    pltpu.sync_copy(
                        srcs_hbm.at[c, s, pl.ds((k + 1) * CH, CH)], src_v)
                    pltpu.sync_copy(
                        dsts_hbm.at[s, pl.ds((k + 1) * CH, CH)], dst_v)
                    for b in range(nbuf):
                        pltpu.async_copy(table_hbm.at[src_v.at[b]],
                                         rows_v[b], sems[b])
                return carry

            lax.fori_loop(0, NCHUNK, chunk, 0)

        stage0_and_prime(tables[0])
        pltpu.sync_copy(zeros_hbm, acc.at[pl.ds(s * ZR, ZR)])
        plsc.subcore_barrier()
        for t in range(ntab):
            run_table(tables[t])
            plsc.subcore_barrier()
            if t + 1 < ntab:
                stage0_and_prime(tables[t + 1])
            pltpu.sync_copy(acc.at[pl.ds(s * ZR, ZR)],
                            outs[t].at[pl.ds(c * NP + s * ZR, ZR)])
            if t + 1 < ntab:
                pltpu.sync_copy(zeros_hbm, acc.at[pl.ds(s * ZR, ZR)])
                plsc.subcore_barrier()

    return sc_agg


def _xprep_body(x_ref, dinv_ref, out_ref):
    xs = x_ref[0] * dinv_ref[...]
    out_ref[0, 0] = xs[:, :128]
    out_ref[0, 1] = xs[:, 128:]


def _cell_body(accx_ref, acch_ref, dinv_ref, c_ref, wx_ref, wh_ref, b_ref,
               h_ref, cn_ref, hp_ref):
    d = dinv_ref[...]
    ax = jnp.concatenate([accx_ref[0], accx_ref[1]], axis=1) * d
    ah = jnp.concatenate([acch_ref[0], acch_ref[1]], axis=1) * d
    gates = (jnp.dot(ax, wx_ref[...], preferred_element_type=jnp.float32)
             + jnp.dot(ah, wh_ref[...], preferred_element_type=jnp.float32)
             + b_ref[...])
    i = jax.nn.sigmoid(gates[:, 0 * HH:1 * HH])
    f = jax.nn.sigmoid(gates[:, 1 * HH:2 * HH])
    g = jnp.tanh(gates[:, 2 * HH:3 * HH])
    o = jax.nn.sigmoid(gates[:, 3 * HH:4 * HH])
    cn = f * c_ref[...] + i * g
    h = o * jnp.tanh(cn)
    h_ref[...] = h
    cn_ref[...] = cn
    hp = h * d
    hp_ref[0, 0] = hp[:, :128]
    hp_ref[0, 1] = hp[:, 128:]


def _cellpair_body(g0_ref, g1_ref, axn_ref, dinv_ref, c1_ref, c0_ref,
                   wx1_ref, wh1_ref, b1_ref, wx0_ref, wh0_ref, b0_ref,
                   h1_ref, c1n_ref, hp1_ref, c0n_ref, hp0_ref):
    d = dinv_ref[...]
    g0 = jnp.concatenate([g0_ref[0], g0_ref[1]], axis=1) * d

    ah1 = jnp.concatenate([g1_ref[0], g1_ref[1]], axis=1) * d
    gates1 = (jnp.dot(g0, wx1_ref[...], preferred_element_type=jnp.float32)
              + jnp.dot(ah1, wh1_ref[...], preferred_element_type=jnp.float32)
              + b1_ref[...])
    i1 = jax.nn.sigmoid(gates1[:, 0 * HH:1 * HH])
    f1 = jax.nn.sigmoid(gates1[:, 1 * HH:2 * HH])
    gg1 = jnp.tanh(gates1[:, 2 * HH:3 * HH])
    o1 = jax.nn.sigmoid(gates1[:, 3 * HH:4 * HH])
    cn1 = f1 * c1_ref[...] + i1 * gg1
    h1 = o1 * jnp.tanh(cn1)
    h1_ref[...] = h1
    c1n_ref[...] = cn1
    hp1 = h1 * d
    hp1_ref[0, 0] = hp1[:, :128]
    hp1_ref[0, 1] = hp1[:, 128:]

    ax0 = jnp.concatenate([axn_ref[0], axn_ref[1]], axis=1) * d
    gates0 = (jnp.dot(ax0, wx0_ref[...], preferred_element_type=jnp.float32)
              + jnp.dot(g0, wh0_ref[...], preferred_element_type=jnp.float32)
              + b0_ref[...])
    i0 = jax.nn.sigmoid(gates0[:, 0 * HH:1 * HH])
    f0 = jax.nn.sigmoid(gates0[:, 1 * HH:2 * HH])
    gg0 = jnp.tanh(gates0[:, 2 * HH:3 * HH])
    o0 = jax.nn.sigmoid(gates0[:, 3 * HH:4 * HH])
    cn0 = f0 * c0_ref[...] + i0 * gg0
    h0 = o0 * jnp.tanh(cn0)
    c0n_ref[...] = cn0
    hp0 = h0 * d
    hp0_ref[0, 0] = hp0[:, :128]
    hp0_ref[0, 1] = hp0[:, 128:]


def _fc_body(h_ref, w_ref, b_ref, o_ref):
    o_ref[...] = jax.nn.sigmoid(
        jnp.dot(h_ref[...], w_ref[...], preferred_element_type=jnp.float32)
        + b_ref[...])


_BM = 256

_cell_call = pl.pallas_call(
    _cell_body,
    grid=(NP // _BM,),
    in_specs=[
        pl.BlockSpec((2, _BM, 128), lambda n: (0, n, 0)),
        pl.BlockSpec((2, _BM, 128), lambda n: (0, n, 0)),
        pl.BlockSpec((_BM, 1), lambda n: (n, 0)),
        pl.BlockSpec((_BM, HH), lambda n: (n, 0)),
        pl.BlockSpec((DD, 4 * HH), lambda n: (0, 0)),
        pl.BlockSpec((HH, 4 * HH), lambda n: (0, 0)),
        pl.BlockSpec((1, 4 * HH), lambda n: (0, 0)),
    ],
    out_specs=[
        pl.BlockSpec((_BM, HH), lambda n: (n, 0)),
        pl.BlockSpec((_BM, HH), lambda n: (n, 0)),
        pl.BlockSpec((1, 2, _BM, 128), lambda n: (0, 0, n, 0)),
    ],
    out_shape=[
        jax.ShapeDtypeStruct((NP, HH), jnp.float32),
        jax.ShapeDtypeStruct((NP, HH), jnp.float32),
        jax.ShapeDtypeStruct((1, 2, NP, 128), jnp.float32),
    ],
)

_wspec = pl.BlockSpec((DD, 4 * HH), lambda n: (0, 0))
_bspec = pl.BlockSpec((1, 4 * HH), lambda n: (0, 0))

_cellpair_call = pl.pallas_call(
    _cellpair_body,
    grid=(NP // _BM,),
    in_specs=[
        pl.BlockSpec((2, _BM, 128), lambda n: (0, n, 0)),
        pl.BlockSpec((2, _BM, 128), lambda n: (0, n, 0)),
        pl.BlockSpec((2, _BM, 128), lambda n: (0, n, 0)),
        pl.BlockSpec((_BM, 1), lambda n: (n, 0)),
        pl.BlockSpec((_BM, HH), lambda n: (n, 0)),
        pl.BlockSpec((_BM, HH), lambda n: (n, 0)),
        _wspec, _wspec, _bspec,
        _wspec, _wspec, _bspec,
    ],
    out_specs=[
        pl.BlockSpec((_BM, HH), lambda n: (n, 0)),
        pl.BlockSpec((_BM, HH), lambda n: (n, 0)),
        pl.BlockSpec((1, 2, _BM, 128), lambda n: (0, 0, n, 0)),
        pl.BlockSpec((_BM, HH), lambda n: (n, 0)),
        pl.BlockSpec((1, 2, _BM, 128), lambda n: (0, 0, n, 0)),
    ],
    out_shape=[
        jax.ShapeDtypeStruct((NP, HH), jnp.float32),
        jax.ShapeDtypeStruct((NP, HH), jnp.float32),
        jax.ShapeDtypeStruct((1, 2, NP, 128), jnp.float32),
        jax.ShapeDtypeStruct((NP, HH), jnp.float32),
        jax.ShapeDtypeStruct((1, 2, NP, 128), jnp.float32),
    ],
)

_xprep_call = pl.pallas_call(
    _xprep_body,
    grid=(TT, NP // _BM),
    in_specs=[
        pl.BlockSpec((1, _BM, DD), lambda t, n: (t, n, 0)),
        pl.BlockSpec((_BM, 1), lambda t, n: (n, 0)),
    ],
    out_specs=pl.BlockSpec((1, 2, _BM, 128), lambda t, n: (t, 0, n, 0)),
    out_shape=jax.ShapeDtypeStruct((TT, 2, NP, 128), jnp.float32),
)

_fc_call = pl.pallas_call(
    _fc_body,
    out_shape=jax.ShapeDtypeStruct((NP, 128), jnp.float32),
)


def kernel(x, edge_index, Wx0, Wh0, b0, Wx1, Wh1, b1, Wfc, bfc):
    src = edge_index[0].astype(jnp.int32)
    dst = edge_index[1].astype(jnp.int32)

    pad = EP - EE
    src_p = jnp.concatenate([src, jnp.full((pad,), NN, jnp.int32)])
    dst_p = jnp.concatenate([dst, jnp.full((pad,), NN, jnp.int32)])
    srcs = jnp.stack([src_p, src_p + NP]).reshape(NC, NS, NB, BB)
    dsts = dst_p.reshape(NS, NB, BB)

    zeros_agg = jnp.zeros((ZR, 128), jnp.float32)
    ones_tbl = jnp.ones((2 * NP, 128), jnp.float32)

    sc1 = _sc_agg_multi(1)
    agg = lambda tbl: sc1(tbl, srcs, dsts, zeros_agg)[0].reshape(2, NP, 128)

    (degp,) = sc1(ones_tbl, srcs, dsts, zeros_agg)
    deg = degp[:NP, 0]
    dinv = jax.lax.rsqrt(jnp.clip(deg, 1.0, None)).reshape(NP, 1)

    xpad = jnp.pad(x, ((0, 0), (0, NP - NN), (0, 0)))
    xp = _xprep_call(xpad, dinv).reshape(TT, 2 * NP, 128)

    z2 = jnp.zeros((2, NP, 128), jnp.float32)
    zN = jnp.zeros((NP, HH), jnp.float32)
    b0r = b0.reshape(1, 4 * HH)
    b1r = b1.reshape(1, 4 * HH)

    ax_cur = agg(xp[0])

    _, c0, hp0 = _cell_call(ax_cur, z2, dinv, zN, Wx0, Wh0, b0r)
    g0 = agg(hp0.reshape(2 * NP, 128))
    ax_next = agg(xp[1])

    g1 = z2
    c1 = zN
    h1 = zN
    for t in range(TT - 1):
        h1, c1, hp1, c0, hp0 = _cellpair_call(
            g0, g1, ax_next, dinv, c1, c0, Wx1, Wh1, b1r, Wx0, Wh0, b0r)
        g1 = agg(hp1.reshape(2 * NP, 128))
        g0 = agg(hp0.reshape(2 * NP, 128))
        if t + 2 < TT:
            ax_next = agg(xp[t + 2])
    h1, c1, hp1 = _cell_call(g0, g1, dinv, c1, Wx1, Wh1, b1r)

    Wfc_pad = jnp.pad(Wfc, ((0, 0), (0, 127)))
    bfc_pad = jnp.pad(bfc, ((0, 127))).reshape(1, 128)
    score = _fc_call(h1, Wfc_pad, bfc_pad)
    return score[:NN, :1]


# speedup vs baseline: 1.1187x; 1.1187x over previous
import functools

import jax
import jax.numpy as jnp
from jax import lax
from jax.experimental import pallas as pl
from jax.experimental.pallas import tpu as pltpu
from jax.experimental.pallas import tpu_sc as plsc

NN = 10000
NP = 10240
EE = 160000
TT = 8
DD = 256
HH = 256

NC = 2
NS = 16
BB = 64
NB = 160
EP = NS * NB * BB
ZR = NP // NS


@functools.lru_cache(maxsize=None)
def _sc_agg_multi(ntab):
    mesh = plsc.VectorSubcoreMesh(core_axis_name="c", subcore_axis_name="s",
                                  num_cores=NC, num_subcores=NS)
    nbuf = 5
    CH = 16
    NCHUNK = NB // CH

    @functools.partial(
        pl.kernel,
        out_type=[jax.ShapeDtypeStruct((2 * NP, 128), jnp.float32)] * ntab,
        mesh=mesh,
        scratch_types=[
            pltpu.VMEM((CH, BB), jnp.int32),
            pltpu.VMEM((CH, BB), jnp.int32),
            [pltpu.VMEM((BB, 128), jnp.float32)] * nbuf,
            pltpu.VMEM_SHARED((NP, 128), jnp.float32),
            [pltpu.SemaphoreType.DMA] * nbuf,
        ],
    )
    def sc_agg(*refs):
        tables = refs[:ntab]
        srcs_hbm, dsts_hbm, zeros_hbm = refs[ntab:ntab + 3]
        outs = refs[ntab + 3:2 * ntab + 3]
        src_v, dst_v, rows_v, acc, sems = refs[2 * ntab + 3:]
        c = lax.axis_index("c")
        s = lax.axis_index("s")

        def stage0_and_prime(table_hbm):
            pltpu.sync_copy(srcs_hbm.at[c, s, pl.ds(0, CH)], src_v)
            pltpu.sync_copy(dsts_hbm.at[s, pl.ds(0, CH)], dst_v)
            for b in range(nbuf):
                pltpu.async_copy(table_hbm.at[src_v.at[b]], rows_v[b],
                                 sems[b])

        def run_table(table_hbm):
            def chunk(k, carry):
                def one(j, refire):
                    b = j % nbuf
                    pltpu.make_async_copy(table_hbm.at[src_v.at[j]],
                                          rows_v[b], sems[b]).wait()
                    pltpu.sync_copy(rows_v[b], acc.at[dst_v.at[j]], add=True)
                    if refire:
                        @pl.when(j + nbuf < CH)
                        def _():
                            pltpu.async_copy(
                                table_hbm.at[src_v.at[j + nbuf]],
                                rows_v[b], sems[b])

                for j in range(0, CH - nbuf):
                    one(j, True)
                for j in range(CH - nbuf, CH):
                    one(j, False)

                @pl.when(k + 1 < NCHUNK)
                def _():
                    pltpu.sync_copy(
                        srcs_hbm.at[c, s, pl.ds((k + 1) * CH, CH)], src_v)
                    pltpu.sync_copy(
                        dsts_hbm.at[s, pl.ds((k + 1) * CH, CH)], dst_v)
                    for b in range(nbuf):
                        pltpu.async_copy(table_hbm.at[src_v.at[b]],
                                         rows_v[b], sems[b])
                return carry

            lax.fori_loop(0, NCHUNK, chunk, 0)

        stage0_and_prime(tables[0])
        pltpu.sync_copy(zeros_hbm, acc.at[pl.ds(s * ZR, ZR)])
        plsc.subcore_barrier()
        for t in range(ntab):
            run_table(tables[t])
            plsc.subcore_barrier()
            if t + 1 < ntab:
                stage0_and_prime(tables[t + 1])
            pltpu.sync_copy(acc.at[pl.ds(s * ZR, ZR)],
                            outs[t].at[pl.ds(c * NP + s * ZR, ZR)])
            if t + 1 < ntab:
                pltpu.sync_copy(zeros_hbm, acc.at[pl.ds(s * ZR, ZR)])
                plsc.subcore_barrier()

    return sc_agg


def _xprep_body(x_ref, dinv_ref, out_ref):
    xs = x_ref[0] * dinv_ref[...]
    out_ref[0, 0] = xs[:, :128]
    out_ref[0, 1] = xs[:, 128:]


def _cell_body(accx_ref, acch_ref, dinv_ref, c_ref, wx_ref, wh_ref, b_ref,
               h_ref, cn_ref, hp_ref):
    d = dinv_ref[...]
    ax = jnp.concatenate([accx_ref[0], accx_ref[1]], axis=1) * d
    ah = jnp.concatenate([acch_ref[0], acch_ref[1]], axis=1) * d
    gates = (jnp.dot(ax, wx_ref[...], preferred_element_type=jnp.float32)
             + jnp.dot(ah, wh_ref[...], preferred_element_type=jnp.float32)
             + b_ref[...])
    i = jax.nn.sigmoid(gates[:, 0 * HH:1 * HH])
    f = jax.nn.sigmoid(gates[:, 1 * HH:2 * HH])
    g = jnp.tanh(gates[:, 2 * HH:3 * HH])
    o = jax.nn.sigmoid(gates[:, 3 * HH:4 * HH])
    cn = f * c_ref[...] + i * g
    h = o * jnp.tanh(cn)
    h_ref[...] = h
    cn_ref[...] = cn
    hp = h * d
    hp_ref[0, 0] = hp[:, :128]
    hp_ref[0, 1] = hp[:, 128:]


def _cellpair_body(g0_ref, g1_ref, axn_ref, dinv_ref, c1_ref, c0_ref,
                   wx1_ref, wh1_ref, b1_ref, wx0_ref, wh0_ref, b0_ref,
                   h1_ref, c1n_ref, hp1_ref, c0n_ref, hp0_ref):
    d = dinv_ref[...]
    g0 = jnp.concatenate([g0_ref[0], g0_ref[1]], axis=1) * d

    ah1 = jnp.concatenate([g1_ref[0], g1_ref[1]], axis=1) * d
    gates1 = (jnp.dot(g0, wx1_ref[...], preferred_element_type=jnp.float32)
              + jnp.dot(ah1, wh1_ref[...], preferred_element_type=jnp.float32)
              + b1_ref[...])
    i1 = jax.nn.sigmoid(gates1[:, 0 * HH:1 * HH])
    f1 = jax.nn.sigmoid(gates1[:, 1 * HH:2 * HH])
    gg1 = jnp.tanh(gates1[:, 2 * HH:3 * HH])
    o1 = jax.nn.sigmoid(gates1[:, 3 * HH:4 * HH])
    cn1 = f1 * c1_ref[...] + i1 * gg1
    h1 = o1 * jnp.tanh(cn1)
    h1_ref[...] = h1
    c1n_ref[...] = cn1
    hp1 = h1 * d
    hp1_ref[0, 0] = hp1[:, :128]
    hp1_ref[0, 1] = hp1[:, 128:]

    ax0 = jnp.concatenate([axn_ref[0], axn_ref[1]], axis=1) * d
    gates0 = (jnp.dot(ax0, wx0_ref[...], preferred_element_type=jnp.float32)
              + jnp.dot(g0, wh0_ref[...], preferred_element_type=jnp.float32)
              + b0_ref[...])
    i0 = jax.nn.sigmoid(gates0[:, 0 * HH:1 * HH])
    f0 = jax.nn.sigmoid(gates0[:, 1 * HH:2 * HH])
    gg0 = jnp.tanh(gates0[:, 2 * HH:3 * HH])
    o0 = jax.nn.sigmoid(gates0[:, 3 * HH:4 * HH])
    cn0 = f0 * c0_ref[...] + i0 * gg0
    h0 = o0 * jnp.tanh(cn0)
    c0n_ref[...] = cn0
    hp0 = h0 * d
    hp0_ref[0, 0] = hp0[:, :128]
    hp0_ref[0, 1] = hp0[:, 128:]


def _fc_body(h_ref, w_ref, b_ref, o_ref):
    o_ref[...] = jax.nn.sigmoid(
        jnp.dot(h_ref[...], w_ref[...], preferred_element_type=jnp.float32)
        + b_ref[...])


_BM = 256

_cell_call = pl.pallas_call(
    _cell_body,
    grid=(NP // _BM,),
    in_specs=[
        pl.BlockSpec((2, _BM, 128), lambda n: (0, n, 0)),
        pl.BlockSpec((2, _BM, 128), lambda n: (0, n, 0)),
        pl.BlockSpec((_BM, 1), lambda n: (n, 0)),
        pl.BlockSpec((_BM, HH), lambda n: (n, 0)),
        pl.BlockSpec((DD, 4 * HH), lambda n: (0, 0)),
        pl.BlockSpec((HH, 4 * HH), lambda n: (0, 0)),
        pl.BlockSpec((1, 4 * HH), lambda n: (0, 0)),
    ],
    out_specs=[
        pl.BlockSpec((_BM, HH), lambda n: (n, 0)),
        pl.BlockSpec((_BM, HH), lambda n: (n, 0)),
        pl.BlockSpec((1, 2, _BM, 128), lambda n: (0, 0, n, 0)),
    ],
    out_shape=[
        jax.ShapeDtypeStruct((NP, HH), jnp.float32),
        jax.ShapeDtypeStruct((NP, HH), jnp.float32),
        jax.ShapeDtypeStruct((1, 2, NP, 128), jnp.float32),
    ],
)

_wspec = pl.BlockSpec((DD, 4 * HH), lambda n: (0, 0))
_bspec = pl.BlockSpec((1, 4 * HH), lambda n: (0, 0))

_cellpair_call = pl.pallas_call(
    _cellpair_body,
    grid=(NP // _BM,),
    in_specs=[
        pl.BlockSpec((2, _BM, 128), lambda n: (0, n, 0)),
        pl.BlockSpec((2, _BM, 128), lambda n: (0, n, 0)),
        pl.BlockSpec((2, _BM, 128), lambda n: (0, n, 0)),
        pl.BlockSpec((_BM, 1), lambda n: (n, 0)),
        pl.BlockSpec((_BM, HH), lambda n: (n, 0)),
        pl.BlockSpec((_BM, HH), lambda n: (n, 0)),
        _wspec, _wspec, _bspec,
        _wspec, _wspec, _bspec,
    ],
    out_specs=[
        pl.BlockSpec((_BM, HH), lambda n: (n, 0)),
        pl.BlockSpec((_BM, HH), lambda n: (n, 0)),
        pl.BlockSpec((1, 2, _BM, 128), lambda n: (0, 0, n, 0)),
        pl.BlockSpec((_BM, HH), lambda n: (n, 0)),
        pl.BlockSpec((1, 2, _BM, 128), lambda n: (0, 0, n, 0)),
    ],
    out_shape=[
        jax.ShapeDtypeStruct((NP, HH), jnp.float32),
        jax.ShapeDtypeStruct((NP, HH), jnp.float32),
        jax.ShapeDtypeStruct((1, 2, NP, 128), jnp.float32),
        jax.ShapeDtypeStruct((NP, HH), jnp.float32),
        jax.ShapeDtypeStruct((1, 2, NP, 128), jnp.float32),
    ],
)

_xprep_call = pl.pallas_call(
    _xprep_body,
    grid=(TT, NP // _BM),
    in_specs=[
        pl.BlockSpec((1, _BM, DD), lambda t, n: (t, n, 0)),
        pl.BlockSpec((_BM, 1), lambda t, n: (n, 0)),
    ],
    out_specs=pl.BlockSpec((1, 2, _BM, 128), lambda t, n: (t, 0, n, 0)),
    out_shape=jax.ShapeDtypeStruct((TT, 2, NP, 128), jnp.float32),
)

_fc_call = pl.pallas_call(
    _fc_body,
    out_shape=jax.ShapeDtypeStruct((NP, 128), jnp.float32),
)


def kernel(x, edge_index, Wx0, Wh0, b0, Wx1, Wh1, b1, Wfc, bfc):
    src = edge_index[0].astype(jnp.int32)
    dst = edge_index[1].astype(jnp.int32)

    pad = EP - EE
    src_p = jnp.concatenate([src, jnp.full((pad,), NN, jnp.int32)])
    dst_p = jnp.concatenate([dst, jnp.full((pad,), NN, jnp.int32)])
    srcs = jnp.stack([src_p, src_p + NP]).reshape(NC, NS, NB, BB)
    dsts = dst_p.reshape(NS, NB, BB)

    zeros_agg = jnp.zeros((ZR, 128), jnp.float32)
    ones_tbl = jnp.ones((2 * NP, 128), jnp.float32)

    sc1 = _sc_agg_multi(1)
    agg = lambda tbl: sc1(tbl, srcs, dsts, zeros_agg)[0].reshape(2, NP, 128)

    (degp,) = sc1(ones_tbl, srcs, dsts, zeros_agg)
    deg = degp[:NP, 0]
    dinv = jax.lax.rsqrt(jnp.clip(deg, 1.0, None)).reshape(NP, 1)

    xpad = jnp.pad(x, ((0, 0), (0, NP - NN), (0, 0)))
    xp = _xprep_call(xpad, dinv).reshape(TT, 2 * NP, 128)

    z2 = jnp.zeros((2, NP, 128), jnp.float32)
    zN = jnp.zeros((NP, HH), jnp.float32)
    b0r = b0.reshape(1, 4 * HH)
    b1r = b1.reshape(1, 4 * HH)

    ax_cur = agg(xp[0])

    _, c0, hp0 = _cell_call(ax_cur, z2, dinv, zN, Wx0, Wh0, b0r)
    g0 = agg(hp0.reshape(2 * NP, 128))
    ax_next = agg(xp[1])

    g1 = z2
    c1 = zN
    h1 = zN
    for t in range(TT - 1):
        h1, c1, hp1, c0, hp0 = _cellpair_call(
            g0, g1, ax_next, dinv, c1, c0, Wx1, Wh1, b1r, Wx0, Wh0, b0r)
        g1 = agg(hp1.reshape(2 * NP, 128))
        g0 = agg(hp0.reshape(2 * NP, 128))
        if t + 2 < TT:
            ax_next = agg(xp[t + 2])
    h1, c1, hp1 = _cell_call(g0, g1, dinv, c1, Wx1, Wh1, b1r)

    Wfc_pad = jnp.pad(Wfc, ((0, 0), (0, 127)))
    bfc_pad = jnp.pad(bfc, ((0, 127))).reshape(1, 128)
    score = _fc_call(h1, Wfc_pad, bfc_pad)
    return score[:NN, :1]


# speedup vs baseline: 1.1256x; 1.0062x over previous
import functools

import jax
import jax.numpy as jnp
from jax import lax
from jax.experimental import pallas as pl
from jax.experimental.pallas import tpu as pltpu
from jax.experimental.pallas import tpu_sc as plsc

NN = 10000
NP = 10240
EE = 160000
TT = 8
DD = 256
HH = 256

NC = 2
NS = 16
BB = 64
NB = 160
EP = NS * NB * BB
ZR = NP // NS


@functools.lru_cache(maxsize=None)
def _sc_agg_multi(ntab):
    mesh = plsc.VectorSubcoreMesh(core_axis_name="c", subcore_axis_name="s",
                                  num_cores=NC, num_subcores=NS)
    nbuf = 5
    CH = 16
    NCHUNK = NB // CH

    @functools.partial(
        pl.kernel,
        out_type=[jax.ShapeDtypeStruct((2 * NP, 128), jnp.float32)] * ntab,
        mesh=mesh,
        scratch_types=[
            pltpu.VMEM((CH, BB), jnp.int32),
            pltpu.VMEM((CH, BB), jnp.int32),
            [pltpu.VMEM((BB, 128), jnp.float32)] * nbuf,
            pltpu.VMEM_SHARED((NP, 128), jnp.float32),
            [pltpu.SemaphoreType.DMA] * nbuf,
        ],
    )
    def sc_agg(*refs):
        tables = refs[:ntab]
        srcs_hbm, dsts_hbm, zeros_hbm = refs[ntab:ntab + 3]
        outs = refs[ntab + 3:2 * ntab + 3]
        src_v, dst_v, rows_v, acc, sems = refs[2 * ntab + 3:]
        c = lax.axis_index("c")
        s = lax.axis_index("s")

        def stage0_and_prime(table_hbm):
            pltpu.sync_copy(srcs_hbm.at[c, s, pl.ds(0, CH)], src_v)
            pltpu.sync_copy(dsts_hbm.at[s, pl.ds(0, CH)], dst_v)
            for b in range(nbuf):
                pltpu.async_copy(table_hbm.at[src_v.at[b]], rows_v[b],
                                 sems[b])

        def run_table(table_hbm):
            def chunk(k, carry):
                def one(j, refire):
                    b = j % nbuf
                    pltpu.make_async_copy(table_hbm.at[src_v.at[j]],
                                          rows_v[b], sems[b]).wait()
                    pltpu.sync_copy(rows_v[b], acc.at[dst_v.at[j]], add=True)
                    if refire:
                        @pl.when(j + nbuf < CH)
                        def _():
                            pltpu.async_copy(
                                table_hbm.at[src_v.at[j + nbuf]],
                                rows_v[b], sems[b])

                for j in range(0, CH - nbuf):
                    one(j, True)
                for j in range(CH - nbuf, CH):
                    one(j, False)

                @pl.when(k + 1 < NCHUNK)
                def _():
                    pltpu.sync_copy(
                        srcs_hbm.at[c, s, pl.ds((k + 1) * CH, CH)], src_v)
                    pltpu.sync_copy(
                        dsts_hbm.at[s, pl.ds((k + 1) * CH, CH)], dst_v)
                    for b in range(nbuf):
                        pltpu.async_copy(table_hbm.at[src_v.at[b]],
                                         rows_v[b], sems[b])
                return carry

            lax.fori_loop(0, NCHUNK, chunk, 0)

        stage0_and_prime(tables[0])
        pltpu.sync_copy(zeros_hbm, acc.at[pl.ds(s * ZR, ZR)])
        plsc.subcore_barrier()
        for t in range(ntab):
            run_table(tables[t])
            plsc.subcore_barrier()
            if t + 1 < ntab:
                stage0_and_prime(tables[t + 1])
            pltpu.sync_copy(acc.at[pl.ds(s * ZR, ZR)],
                            outs[t].at[pl.ds(c * NP + s * ZR, ZR)])
            if t + 1 < ntab:
                pltpu.sync_copy(zeros_hbm, acc.at[pl.ds(s * ZR, ZR)])
                plsc.subcore_barrier()

    return sc_agg


def _xprep_body(x_ref, dinv_ref, out_ref):
    xs = x_ref[0] * dinv_ref[...]
    out_ref[0, 0] = xs[:, :128]
    out_ref[0, 1] = xs[:, 128:]


def _cell_body(accx_ref, acch_ref, dinv_ref, c_ref, wx_ref, wh_ref, b_ref,
               h_ref, cn_ref, hp_ref):
    d = dinv_ref[...]
    ax = jnp.concatenate([accx_ref[0], accx_ref[1]], axis=1) * d
    ah = jnp.concatenate([acch_ref[0], acch_ref[1]], axis=1) * d
    gates = (jnp.dot(ax, wx_ref[...], preferred_element_type=jnp.float32)
             + jnp.dot(ah, wh_ref[...], preferred_element_type=jnp.float32)
             + b_ref[...])
    i = jax.nn.sigmoid(gates[:, 0 * HH:1 * HH])
    f = jax.nn.sigmoid(gates[:, 1 * HH:2 * HH])
    g = jnp.tanh(gates[:, 2 * HH:3 * HH])
    o = jax.nn.sigmoid(gates[:, 3 * HH:4 * HH])
    cn = f * c_ref[...] + i * g
    h = o * jnp.tanh(cn)
    h_ref[...] = h
    cn_ref[...] = cn
    hp = h * d
    hp_ref[0, 0] = hp[:, :128]
    hp_ref[0, 1] = hp[:, 128:]


def _cellpair_body(g0_ref, g1_ref, axn_ref, dinv_ref, c1_ref, c0_ref,
                   wx1_ref, wh1_ref, b1_ref, wx0_ref, wh0_ref, b0_ref,
                   h1_ref, c1n_ref, hp1_ref, c0n_ref, hp0_ref):
    d = dinv_ref[...]
    g0 = jnp.concatenate([g0_ref[0], g0_ref[1]], axis=1) * d

    ah1 = jnp.concatenate([g1_ref[0], g1_ref[1]], axis=1) * d
    gates1 = (jnp.dot(g0, wx1_ref[...], preferred_element_type=jnp.float32)
              + jnp.dot(ah1, wh1_ref[...], preferred_element_type=jnp.float32)
              + b1_ref[...])
    i1 = jax.nn.sigmoid(gates1[:, 0 * HH:1 * HH])
    f1 = jax.nn.sigmoid(gates1[:, 1 * HH:2 * HH])
    gg1 = jnp.tanh(gates1[:, 2 * HH:3 * HH])
    o1 = jax.nn.sigmoid(gates1[:, 3 * HH:4 * HH])
    cn1 = f1 * c1_ref[...] + i1 * gg1
    h1 = o1 * jnp.tanh(cn1)
    h1_ref[...] = h1
    c1n_ref[...] = cn1
    hp1 = h1 * d
    hp1_ref[0, 0] = hp1[:, :128]
    hp1_ref[0, 1] = hp1[:, 128:]

    ax0 = jnp.concatenate([axn_ref[0], axn_ref[1]], axis=1) * d
    gates0 = (jnp.dot(ax0, wx0_ref[...], preferred_element_type=jnp.float32)
              + jnp.dot(g0, wh0_ref[...], preferred_element_type=jnp.float32)
              + b0_ref[...])
    i0 = jax.nn.sigmoid(gates0[:, 0 * HH:1 * HH])
    f0 = jax.nn.sigmoid(gates0[:, 1 * HH:2 * HH])
    gg0 = jnp.tanh(gates0[:, 2 * HH:3 * HH])
    o0 = jax.nn.sigmoid(gates0[:, 3 * HH:4 * HH])
    cn0 = f0 * c0_ref[...] + i0 * gg0
    h0 = o0 * jnp.tanh(cn0)
    c0n_ref[...] = cn0
    hp0 = h0 * d
    hp0_ref[0, 0] = hp0[:, :128]
    hp0_ref[0, 1] = hp0[:, 128:]


def _fc_body(h_ref, w_ref, b_ref, o_ref):
    o_ref[...] = jax.nn.sigmoid(
        jnp.dot(h_ref[...], w_ref[...], preferred_element_type=jnp.float32)
        + b_ref[...])


_BM = 256

_cell_call = pl.pallas_call(
    _cell_body,
    grid=(NP // _BM,),
    in_specs=[
        pl.BlockSpec((2, _BM, 128), lambda n: (0, n, 0)),
        pl.BlockSpec((2, _BM, 128), lambda n: (0, n, 0)),
        pl.BlockSpec((_BM, 1), lambda n: (n, 0)),
        pl.BlockSpec((_BM, HH), lambda n: (n, 0)),
        pl.BlockSpec((DD, 4 * HH), lambda n: (0, 0)),
        pl.BlockSpec((HH, 4 * HH), lambda n: (0, 0)),
        pl.BlockSpec((1, 4 * HH), lambda n: (0, 0)),
    ],
    out_specs=[
        pl.BlockSpec((_BM, HH), lambda n: (n, 0)),
        pl.BlockSpec((_BM, HH), lambda n: (n, 0)),
        pl.BlockSpec((1, 2, _BM, 128), lambda n: (0, 0, n, 0)),
    ],
    out_shape=[
        jax.ShapeDtypeStruct((NP, HH), jnp.float32),
        jax.ShapeDtypeStruct((NP, HH), jnp.float32),
        jax.ShapeDtypeStruct((1, 2, NP, 128), jnp.float32),
    ],
)

_wspec = pl.BlockSpec((DD, 4 * HH), lambda n: (0, 0))
_bspec = pl.BlockSpec((1, 4 * HH), lambda n: (0, 0))

_cellpair_call = pl.pallas_call(
    _cellpair_body,
    grid=(NP // _BM,),
    in_specs=[
        pl.BlockSpec((2, _BM, 128), lambda n: (0, n, 0)),
        pl.BlockSpec((2, _BM, 128), lambda n: (0, n, 0)),
        pl.BlockSpec((2, _BM, 128), lambda n: (0, n, 0)),
        pl.BlockSpec((_BM, 1), lambda n: (n, 0)),
        pl.BlockSpec((_BM, HH), lambda n: (n, 0)),
        pl.BlockSpec((_BM, HH), lambda n: (n, 0)),
        _wspec, _wspec, _bspec,
        _wspec, _wspec, _bspec,
    ],
    out_specs=[
        pl.BlockSpec((_BM, HH), lambda n: (n, 0)),
        pl.BlockSpec((_BM, HH), lambda n: (n, 0)),
        pl.BlockSpec((1, 2, _BM, 128), lambda n: (0, 0, n, 0)),
        pl.BlockSpec((_BM, HH), lambda n: (n, 0)),
        pl.BlockSpec((1, 2, _BM, 128), lambda n: (0, 0, n, 0)),
    ],
    out_shape=[
        jax.ShapeDtypeStruct((NP, HH), jnp.float32),
        jax.ShapeDtypeStruct((NP, HH), jnp.float32),
        jax.ShapeDtypeStruct((1, 2, NP, 128), jnp.float32),
        jax.ShapeDtypeStruct((NP, HH), jnp.float32),
        jax.ShapeDtypeStruct((1, 2, NP, 128), jnp.float32),
    ],
)

_xprep_call = pl.pallas_call(
    _xprep_body,
    grid=(TT, NP // _BM),
    in_specs=[
        pl.BlockSpec((1, _BM, DD), lambda t, n: (t, n, 0)),
        pl.BlockSpec((_BM, 1), lambda t, n: (n, 0)),
    ],
    out_specs=pl.BlockSpec((1, 2, _BM, 128), lambda t, n: (t, 0, n, 0)),
    out_shape=jax.ShapeDtypeStruct((TT, 2, NP, 128), jnp.float32),
)

_fc_call = pl.pallas_call(
    _fc_body,
    out_shape=jax.ShapeDtypeStruct((NP, 128), jnp.float32),
)


def kernel(x, edge_index, Wx0, Wh0, b0, Wx1, Wh1, b1, Wfc, bfc):
    src = edge_index[0].astype(jnp.int32)
    dst = edge_index[1].astype(jnp.int32)

    pad = EP - EE
    src_p = jnp.concatenate([src, jnp.full((pad,), NN, jnp.int32)])
    dst_p = jnp.concatenate([dst, jnp.full((pad,), NN, jnp.int32)])
    srcs = jnp.stack([src_p, src_p + NP]).reshape(NC, NS, NB, BB)
    dsts = dst_p.reshape(NS, NB, BB)

    zeros_agg = jnp.zeros((ZR, 128), jnp.float32)
    ones_tbl = jnp.ones((2 * NP, 128), jnp.float32)

    sc1 = _sc_agg_multi(1)
    agg = lambda tbl: sc1(tbl, srcs, dsts, zeros_agg)[0].reshape(2, NP, 128)

    (degp,) = sc1(ones_tbl, srcs, dsts, zeros_agg)
    deg = degp[:NP, 0]
    dinv = jax.lax.rsqrt(jnp.clip(deg, 1.0, None)).reshape(NP, 1)

    xpad = jnp.pad(x, ((0, 0), (0, NP - NN), (0, 0)))
    xp = _xprep_call(xpad, dinv).reshape(TT, 2 * NP, 128)

    z2 = jnp.zeros((2, NP, 128), jnp.float32)
    zN = jnp.zeros((NP, HH), jnp.float32)
    b0r = b0.reshape(1, 4 * HH)
    b1r = b1.reshape(1, 4 * HH)

    ax = [agg(xp[t]) for t in range(TT)]

    g0 = z2
    g1 = z2
    c0 = zN
    c1 = zN
    h1 = zN
    for t in range(TT):
        _, c0, hp0 = _cell_call(ax[t], g0, dinv, c0, Wx0, Wh0, b0r)
        g0 = agg(hp0.reshape(2 * NP, 128))
        h1, c1, hp1 = _cell_call(g0, g1, dinv, c1, Wx1, Wh1, b1r)
        if t < TT - 1:
            g1 = agg(hp1.reshape(2 * NP, 128))

    Wfc_pad = jnp.pad(Wfc, ((0, 0), (0, 127)))
    bfc_pad = jnp.pad(bfc, ((0, 127))).reshape(1, 128)
    score = _fc_call(h1, Wfc_pad, bfc_pad)
    return score[:NN, :1]


# speedup vs baseline: 1.1531x; 1.0244x over previous
import functools

import jax
import jax.numpy as jnp
from jax import lax
from jax.experimental import pallas as pl
from jax.experimental.pallas import tpu as pltpu
from jax.experimental.pallas import tpu_sc as plsc

NN = 10000
NP = 10240
EE = 160000
TT = 8
DD = 256
HH = 256

NC = 2
NS = 16
BB = 64
NB = 160
EP = NS * NB * BB
ZR = NP // NS


@functools.lru_cache(maxsize=None)
def _sc_agg_multi(ntab):
    mesh = plsc.VectorSubcoreMesh(core_axis_name="c", subcore_axis_name="s",
                                  num_cores=NC, num_subcores=NS)
    nbuf = 4
    CH = 32
    NCHUNK = NB // CH

    @functools.partial(
        pl.kernel,
        out_type=[jax.ShapeDtypeStruct((2 * NP, 128), jnp.float32)] * ntab,
        mesh=mesh,
        scratch_types=[
            pltpu.VMEM((CH, BB), jnp.int32),
            pltpu.VMEM((CH, BB), jnp.int32),
            [pltpu.VMEM((BB, 128), jnp.float32)] * nbuf,
            pltpu.VMEM_SHARED((NP, 128), jnp.float32),
            [pltpu.SemaphoreType.DMA] * nbuf,
        ],
    )
    def sc_agg(*refs):
        tables = refs[:ntab]
        srcs_hbm, dsts_hbm, zeros_hbm = refs[ntab:ntab + 3]
        outs = refs[ntab + 3:2 * ntab + 3]
        src_v, dst_v, rows_v, acc, sems = refs[2 * ntab + 3:]
        c = lax.axis_index("c")
        s = lax.axis_index("s")

        def stage0_and_prime(table_hbm):
            pltpu.sync_copy(srcs_hbm.at[c, s, pl.ds(0, CH)], src_v)
            pltpu.sync_copy(dsts_hbm.at[s, pl.ds(0, CH)], dst_v)
            for b in range(nbuf):
                pltpu.async_copy(table_hbm.at[src_v.at[b]], rows_v[b],
                                 sems[b])

        def run_table(table_hbm):
            def chunk(k, carry):
                def one(j, refire):
                    b = j % nbuf
                    pltpu.make_async_copy(table_hbm.at[src_v.at[j]],
                                          rows_v[b], sems[b]).wait()
                    pltpu.sync_copy(rows_v[b], acc.at[dst_v.at[j]], add=True)
                    if refire:
                        @pl.when(j + nbuf < CH)
                        def _():
                            pltpu.async_copy(
                                table_hbm.at[src_v.at[j + nbuf]],
                                rows_v[b], sems[b])

                for j in range(0, CH - nbuf):
                    one(j, True)
                for j in range(CH - nbuf, CH):
                    one(j, False)

                @pl.when(k + 1 < NCHUNK)
                def _():
                    pltpu.sync_copy(
                        srcs_hbm.at[c, s, pl.ds((k + 1) * CH, CH)], src_v)
                    pltpu.sync_copy(
                        dsts_hbm.at[s, pl.ds((k + 1) * CH, CH)], dst_v)
                    for b in range(nbuf):
                        pltpu.async_copy(table_hbm.at[src_v.at[b]],
                                         rows_v[b], sems[b])
                return carry

            lax.fori_loop(0, NCHUNK, chunk, 0)

        stage0_and_prime(tables[0])
        pltpu.sync_copy(zeros_hbm, acc.at[pl.ds(s * ZR, ZR)])
        plsc.subcore_barrier()
        for t in range(ntab):
            run_table(tables[t])
            plsc.subcore_barrier()
            if t + 1 < ntab:
                stage0_and_prime(tables[t + 1])
            pltpu.sync_copy(acc.at[pl.ds(s * ZR, ZR)],
                            outs[t].at[pl.ds(c * NP + s * ZR, ZR)])
            if t + 1 < ntab:
                pltpu.sync_copy(zeros_hbm, acc.at[pl.ds(s * ZR, ZR)])
                plsc.subcore_barrier()

    return sc_agg


def _xprep_body(x_ref, dinv_ref, out_ref):
    xs = x_ref[0] * dinv_ref[...]
    out_ref[0, 0] = xs[:, :128]
    out_ref[0, 1] = xs[:, 128:]


def _cell_body(accx_ref, acch_ref, dinv_ref, c_ref, wx_ref, wh_ref, b_ref,
               h_ref, cn_ref, hp_ref):
    d = dinv_ref[...]
    ax = jnp.concatenate([accx_ref[0], accx_ref[1]], axis=1) * d
    ah = jnp.concatenate([acch_ref[0], acch_ref[1]], axis=1) * d
    gates = (jnp.dot(ax, wx_ref[...], preferred_element_type=jnp.float32)
             + jnp.dot(ah, wh_ref[...], preferred_element_type=jnp.float32)
             + b_ref[...])
    i = jax.nn.sigmoid(gates[:, 0 * HH:1 * HH])
    f = jax.nn.sigmoid(gates[:, 1 * HH:2 * HH])
    g = jnp.tanh(gates[:, 2 * HH:3 * HH])
    o = jax.nn.sigmoid(gates[:, 3 * HH:4 * HH])
    cn = f * c_ref[...] + i * g
    h = o * jnp.tanh(cn)
    h_ref[...] = h
    cn_ref[...] = cn
    hp = h * d
    hp_ref[0, 0] = hp[:, :128]
    hp_ref[0, 1] = hp[:, 128:]


def _cellpair_body(g0_ref, g1_ref, axn_ref, dinv_ref, c1_ref, c0_ref,
                   wx1_ref, wh1_ref, b1_ref, wx0_ref, wh0_ref, b0_ref,
                   h1_ref, c1n_ref, hp1_ref, c0n_ref, hp0_ref):
    d = dinv_ref[...]
    g0 = jnp.concatenate([g0_ref[0], g0_ref[1]], axis=1) * d

    ah1 = jnp.concatenate([g1_ref[0], g1_ref[1]], axis=1) * d
    gates1 = (jnp.dot(g0, wx1_ref[...], preferred_element_type=jnp.float32)
              + jnp.dot(ah1, wh1_ref[...], preferred_element_type=jnp.float32)
              + b1_ref[...])
    i1 = jax.nn.sigmoid(gates1[:, 0 * HH:1 * HH])
    f1 = jax.nn.sigmoid(gates1[:, 1 * HH:2 * HH])
    gg1 = jnp.tanh(gates1[:, 2 * HH:3 * HH])
    o1 = jax.nn.sigmoid(gates1[:, 3 * HH:4 * HH])
    cn1 = f1 * c1_ref[...] + i1 * gg1
    h1 = o1 * jnp.tanh(cn1)
    h1_ref[...] = h1
    c1n_ref[...] = cn1
    hp1 = h1 * d
    hp1_ref[0, 0] = hp1[:, :128]
    hp1_ref[0, 1] = hp1[:, 128:]

    ax0 = jnp.concatenate([axn_ref[0], axn_ref[1]], axis=1) * d
    gates0 = (jnp.dot(ax0, wx0_ref[...], preferred_element_type=jnp.float32)
              + jnp.dot(g0, wh0_ref[...], preferred_element_type=jnp.float32)
              + b0_ref[...])
    i0 = jax.nn.sigmoid(gates0[:, 0 * HH:1 * HH])
    f0 = jax.nn.sigmoid(gates0[:, 1 * HH:2 * HH])
    gg0 = jnp.tanh(gates0[:, 2 * HH:3 * HH])
    o0 = jax.nn.sigmoid(gates0[:, 3 * HH:4 * HH])
    cn0 = f0 * c0_ref[...] + i0 * gg0
    h0 = o0 * jnp.tanh(cn0)
    c0n_ref[...] = cn0
    hp0 = h0 * d
    hp0_ref[0, 0] = hp0[:, :128]
    hp0_ref[0, 1] = hp0[:, 128:]


def _fc_body(h_ref, w_ref, b_ref, o_ref):
    o_ref[...] = jax.nn.sigmoid(
        jnp.dot(h_ref[...], w_ref[...], preferred_element_type=jnp.float32)
        + b_ref[...])


_BM = 256

_cell_call = pl.pallas_call(
    _cell_body,
    grid=(NP // _BM,),
    in_specs=[
        pl.BlockSpec((2, _BM, 128), lambda n: (0, n, 0)),
        pl.BlockSpec((2, _BM, 128), lambda n: (0, n, 0)),
        pl.BlockSpec((_BM, 1), lambda n: (n, 0)),
        pl.BlockSpec((_BM, HH), lambda n: (n, 0)),
        pl.BlockSpec((DD, 4 * HH), lambda n: (0, 0)),
        pl.BlockSpec((HH, 4 * HH), lambda n: (0, 0)),
        pl.BlockSpec((1, 4 * HH), lambda n: (0, 0)),
    ],
    out_specs=[
        pl.BlockSpec((_BM, HH), lambda n: (n, 0)),
        pl.BlockSpec((_BM, HH), lambda n: (n, 0)),
        pl.BlockSpec((1, 2, _BM, 128), lambda n: (0, 0, n, 0)),
    ],
    out_shape=[
        jax.ShapeDtypeStruct((NP, HH), jnp.float32),
        jax.ShapeDtypeStruct((NP, HH), jnp.float32),
        jax.ShapeDtypeStruct((1, 2, NP, 128), jnp.float32),
    ],
)

_wspec = pl.BlockSpec((DD, 4 * HH), lambda n: (0, 0))
_bspec = pl.BlockSpec((1, 4 * HH), lambda n: (0, 0))

_cellpair_call = pl.pallas_call(
    _cellpair_body,
    grid=(NP // _BM,),
    in_specs=[
        pl.BlockSpec((2, _BM, 128), lambda n: (0, n, 0)),
        pl.BlockSpec((2, _BM, 128), lambda n: (0, n, 0)),
        pl.BlockSpec((2, _BM, 128), lambda n: (0, n, 0)),
        pl.BlockSpec((_BM, 1), lambda n: (n, 0)),
        pl.BlockSpec((_BM, HH), lambda n: (n, 0)),
        pl.BlockSpec((_BM, HH), lambda n: (n, 0)),
        _wspec, _wspec, _bspec,
        _wspec, _wspec, _bspec,
    ],
    out_specs=[
        pl.BlockSpec((_BM, HH), lambda n: (n, 0)),
        pl.BlockSpec((_BM, HH), lambda n: (n, 0)),
        pl.BlockSpec((1, 2, _BM, 128), lambda n: (0, 0, n, 0)),
        pl.BlockSpec((_BM, HH), lambda n: (n, 0)),
        pl.BlockSpec((1, 2, _BM, 128), lambda n: (0, 0, n, 0)),
    ],
    out_shape=[
        jax.ShapeDtypeStruct((NP, HH), jnp.float32),
        jax.ShapeDtypeStruct((NP, HH), jnp.float32),
        jax.ShapeDtypeStruct((1, 2, NP, 128), jnp.float32),
        jax.ShapeDtypeStruct((NP, HH), jnp.float32),
        jax.ShapeDtypeStruct((1, 2, NP, 128), jnp.float32),
    ],
)

_xprep_call = pl.pallas_call(
    _xprep_body,
    grid=(TT, NP // _BM),
    in_specs=[
        pl.BlockSpec((1, _BM, DD), lambda t, n: (t, n, 0)),
        pl.BlockSpec((_BM, 1), lambda t, n: (n, 0)),
    ],
    out_specs=pl.BlockSpec((1, 2, _BM, 128), lambda t, n: (t, 0, n, 0)),
    out_shape=jax.ShapeDtypeStruct((TT, 2, NP, 128), jnp.float32),
)

_fc_call = pl.pallas_call(
    _fc_body,
    out_shape=jax.ShapeDtypeStruct((NP, 128), jnp.float32),
)


def kernel(x, edge_index, Wx0, Wh0, b0, Wx1, Wh1, b1, Wfc, bfc):
    src = edge_index[0].astype(jnp.int32)
    dst = edge_index[1].astype(jnp.int32)

    pad = EP - EE
    src_p = jnp.concatenate([src, jnp.full((pad,), NN, jnp.int32)])
    dst_p = jnp.concatenate([dst, jnp.full((pad,), NN, jnp.int32)])
    srcs = jnp.stack([src_p, src_p + NP]).reshape(NC, NS, NB, BB)
    dsts = dst_p.reshape(NS, NB, BB)

    zeros_agg = jnp.zeros((ZR, 128), jnp.float32)
    ones_tbl = jnp.ones((2 * NP, 128), jnp.float32)

    sc1 = _sc_agg_multi(1)
    agg = lambda tbl: sc1(tbl, srcs, dsts, zeros_agg)[0].reshape(2, NP, 128)

    (degp,) = sc1(ones_tbl, srcs, dsts, zeros_agg)
    deg = degp[:NP, 0]
    dinv = jax.lax.rsqrt(jnp.clip(deg, 1.0, None)).reshape(NP, 1)

    xpad = jnp.pad(x, ((0, 0), (0, NP - NN), (0, 0)))
    xp = _xprep_call(xpad, dinv).reshape(TT, 2 * NP, 128)

    z2 = jnp.zeros((2, NP, 128), jnp.float32)
    zN = jnp.zeros((NP, HH), jnp.float32)
    b0r = b0.reshape(1, 4 * HH)
    b1r = b1.reshape(1, 4 * HH)

    ax = [agg(xp[t]) for t in range(TT)]

    g0 = z2
    g1 = z2
    c0 = zN
    c1 = zN
    h1 = zN
    for t in range(TT):
        _, c0, hp0 = _cell_call(ax[t], g0, dinv, c0, Wx0, Wh0, b0r)
        g0 = agg(hp0.reshape(2 * NP, 128))
        h1, c1, hp1 = _cell_call(g0, g1, dinv, c1, Wx1, Wh1, b1r)
        if t < TT - 1:
            g1 = agg(hp1.reshape(2 * NP, 128))

    Wfc_pad = jnp.pad(Wfc, ((0, 0), (0, 127)))
    bfc_pad = jnp.pad(bfc, ((0, 127))).reshape(1, 128)
    score = _fc_call(h1, Wfc_pad, bfc_pad)
    return score[:NN, :1]


# speedup vs baseline: 1.1604x; 1.0064x over previous
import functools

import jax
import jax.numpy as jnp
from jax import lax
from jax.experimental import pallas as pl
from jax.experimental.pallas import tpu as pltpu
from jax.experimental.pallas import tpu_sc as plsc

NN = 10000
NP = 10240
EE = 160000
TT = 8
DD = 256
HH = 256

NC = 2
NS = 16
BB = 64
NB = 160
EP = NS * NB * BB
ZR = NP // NS


@functools.lru_cache(maxsize=None)
def _sc_agg_multi(ntab):
    mesh = plsc.VectorSubcoreMesh(core_axis_name="c", subcore_axis_name="s",
                                  num_cores=NC, num_subcores=NS)
    nbuf = 4
    CH = 40
    NCHUNK = NB // CH

    @functools.partial(
        pl.kernel,
        out_type=[jax.ShapeDtypeStruct((2 * NP, 128), jnp.float32)] * ntab,
        mesh=mesh,
        scratch_types=[
            pltpu.VMEM((CH, BB), jnp.int32),
            pltpu.VMEM((CH, BB), jnp.int32),
            [pltpu.VMEM((BB, 128), jnp.float32)] * nbuf,
            pltpu.VMEM_SHARED((NP, 128), jnp.float32),
            [pltpu.SemaphoreType.DMA] * nbuf,
        ],
    )
    def sc_agg(*refs):
        tables = refs[:ntab]
        srcs_hbm, dsts_hbm, zeros_hbm = refs[ntab:ntab + 3]
        outs = refs[ntab + 3:2 * ntab + 3]
        src_v, dst_v, rows_v, acc, sems = refs[2 * ntab + 3:]
        c = lax.axis_index("c")
        s = lax.axis_index("s")

        def stage0_and_prime(table_hbm):
            pltpu.sync_copy(srcs_hbm.at[c, s, pl.ds(0, CH)], src_v)
            pltpu.sync_copy(dsts_hbm.at[s, pl.ds(0, CH)], dst_v)
            for b in range(nbuf):
                pltpu.async_copy(table_hbm.at[src_v.at[b]], rows_v[b],
                                 sems[b])

        def run_table(table_hbm):
            def chunk(k, carry):
                def one(j, refire):
                    b = j % nbuf
                    pltpu.make_async_copy(table_hbm.at[src_v.at[j]],
                                          rows_v[b], sems[b]).wait()
                    pltpu.sync_copy(rows_v[b], acc.at[dst_v.at[j]], add=True)
                    if refire:
                        @pl.when(j + nbuf < CH)
                        def _():
                            pltpu.async_copy(
                                table_hbm.at[src_v.at[j + nbuf]],
                                rows_v[b], sems[b])

                for j in range(0, CH - nbuf):
                    one(j, True)
                for j in range(CH - nbuf, CH):
                    one(j, False)

                @pl.when(k + 1 < NCHUNK)
                def _():
                    pltpu.sync_copy(
                        srcs_hbm.at[c, s, pl.ds((k + 1) * CH, CH)], src_v)
                    pltpu.sync_copy(
                        dsts_hbm.at[s, pl.ds((k + 1) * CH, CH)], dst_v)
                    for b in range(nbuf):
                        pltpu.async_copy(table_hbm.at[src_v.at[b]],
                                         rows_v[b], sems[b])
                return carry

            lax.fori_loop(0, NCHUNK, chunk, 0)

        stage0_and_prime(tables[0])
        pltpu.sync_copy(zeros_hbm, acc.at[pl.ds(s * ZR, ZR)])
        plsc.subcore_barrier()
        for t in range(ntab):
            run_table(tables[t])
            plsc.subcore_barrier()
            if t + 1 < ntab:
                stage0_and_prime(tables[t + 1])
            pltpu.sync_copy(acc.at[pl.ds(s * ZR, ZR)],
                            outs[t].at[pl.ds(c * NP + s * ZR, ZR)])
            if t + 1 < ntab:
                pltpu.sync_copy(zeros_hbm, acc.at[pl.ds(s * ZR, ZR)])
                plsc.subcore_barrier()

    return sc_agg


def _xprep_body(x_ref, dinv_ref, out_ref):
    xs = x_ref[0] * dinv_ref[...]
    out_ref[0, 0] = xs[:, :128]
    out_ref[0, 1] = xs[:, 128:]


def _cell_body(accx_ref, acch_ref, dinv_ref, c_ref, wx_ref, wh_ref, b_ref,
               h_ref, cn_ref, hp_ref):
    d = dinv_ref[...]
    ax = jnp.concatenate([accx_ref[0], accx_ref[1]], axis=1) * d
    ah = jnp.concatenate([acch_ref[0], acch_ref[1]], axis=1) * d
    gates = (jnp.dot(ax, wx_ref[...], preferred_element_type=jnp.float32)
             + jnp.dot(ah, wh_ref[...], preferred_element_type=jnp.float32)
             + b_ref[...])
    i = jax.nn.sigmoid(gates[:, 0 * HH:1 * HH])
    f = jax.nn.sigmoid(gates[:, 1 * HH:2 * HH])
    g = jnp.tanh(gates[:, 2 * HH:3 * HH])
    o = jax.nn.sigmoid(gates[:, 3 * HH:4 * HH])
    cn = f * c_ref[...] + i * g
    h = o * jnp.tanh(cn)
    h_ref[...] = h
    cn_ref[...] = cn
    hp = h * d
    hp_ref[0, 0] = hp[:, :128]
    hp_ref[0, 1] = hp[:, 128:]


def _cellpair_body(g0_ref, g1_ref, axn_ref, dinv_ref, c1_ref, c0_ref,
                   wx1_ref, wh1_ref, b1_ref, wx0_ref, wh0_ref, b0_ref,
                   h1_ref, c1n_ref, hp1_ref, c0n_ref, hp0_ref):
    d = dinv_ref[...]
    g0 = jnp.concatenate([g0_ref[0], g0_ref[1]], axis=1) * d

    ah1 = jnp.concatenate([g1_ref[0], g1_ref[1]], axis=1) * d
    gates1 = (jnp.dot(g0, wx1_ref[...], preferred_element_type=jnp.float32)
              + jnp.dot(ah1, wh1_ref[...], preferred_element_type=jnp.float32)
              + b1_ref[...])
    i1 = jax.nn.sigmoid(gates1[:, 0 * HH:1 * HH])
    f1 = jax.nn.sigmoid(gates1[:, 1 * HH:2 * HH])
    gg1 = jnp.tanh(gates1[:, 2 * HH:3 * HH])
    o1 = jax.nn.sigmoid(gates1[:, 3 * HH:4 * HH])
    cn1 = f1 * c1_ref[...] + i1 * gg1
    h1 = o1 * jnp.tanh(cn1)
    h1_ref[...] = h1
    c1n_ref[...] = cn1
    hp1 = h1 * d
    hp1_ref[0, 0] = hp1[:, :128]
    hp1_ref[0, 1] = hp1[:, 128:]

    ax0 = jnp.concatenate([axn_ref[0], axn_ref[1]], axis=1) * d
    gates0 = (jnp.dot(ax0, wx0_ref[...], preferred_element_type=jnp.float32)
              + jnp.dot(g0, wh0_ref[...], preferred_element_type=jnp.float32)
              + b0_ref[...])
    i0 = jax.nn.sigmoid(gates0[:, 0 * HH:1 * HH])
    f0 = jax.nn.sigmoid(gates0[:, 1 * HH:2 * HH])
    gg0 = jnp.tanh(gates0[:, 2 * HH:3 * HH])
    o0 = jax.nn.sigmoid(gates0[:, 3 * HH:4 * HH])
    cn0 = f0 * c0_ref[...] + i0 * gg0
    h0 = o0 * jnp.tanh(cn0)
    c0n_ref[...] = cn0
    hp0 = h0 * d
    hp0_ref[0, 0] = hp0[:, :128]
    hp0_ref[0, 1] = hp0[:, 128:]


def _fc_body(h_ref, w_ref, b_ref, o_ref):
    o_ref[...] = jax.nn.sigmoid(
        jnp.dot(h_ref[...], w_ref[...], preferred_element_type=jnp.float32)
        + b_ref[...])


_BM = 256

_cell_call = pl.pallas_call(
    _cell_body,
    grid=(NP // _BM,),
    in_specs=[
        pl.BlockSpec((2, _BM, 128), lambda n: (0, n, 0)),
        pl.BlockSpec((2, _BM, 128), lambda n: (0, n, 0)),
        pl.BlockSpec((_BM, 1), lambda n: (n, 0)),
        pl.BlockSpec((_BM, HH), lambda n: (n, 0)),
        pl.BlockSpec((DD, 4 * HH), lambda n: (0, 0)),
        pl.BlockSpec((HH, 4 * HH), lambda n: (0, 0)),
        pl.BlockSpec((1, 4 * HH), lambda n: (0, 0)),
    ],
    out_specs=[
        pl.BlockSpec((_BM, HH), lambda n: (n, 0)),
        pl.BlockSpec((_BM, HH), lambda n: (n, 0)),
        pl.BlockSpec((1, 2, _BM, 128), lambda n: (0, 0, n, 0)),
    ],
    out_shape=[
        jax.ShapeDtypeStruct((NP, HH), jnp.float32),
        jax.ShapeDtypeStruct((NP, HH), jnp.float32),
        jax.ShapeDtypeStruct((1, 2, NP, 128), jnp.float32),
    ],
)

_wspec = pl.BlockSpec((DD, 4 * HH), lambda n: (0, 0))
_bspec = pl.BlockSpec((1, 4 * HH), lambda n: (0, 0))

_cellpair_call = pl.pallas_call(
    _cellpair_body,
    grid=(NP // _BM,),
    in_specs=[
        pl.BlockSpec((2, _BM, 128), lambda n: (0, n, 0)),
        pl.BlockSpec((2, _BM, 128), lambda n: (0, n, 0)),
        pl.BlockSpec((2, _BM, 128), lambda n: (0, n, 0)),
        pl.BlockSpec((_BM, 1), lambda n: (n, 0)),
        pl.BlockSpec((_BM, HH), lambda n: (n, 0)),
        pl.BlockSpec((_BM, HH), lambda n: (n, 0)),
        _wspec, _wspec, _bspec,
        _wspec, _wspec, _bspec,
    ],
    out_specs=[
        pl.BlockSpec((_BM, HH), lambda n: (n, 0)),
        pl.BlockSpec((_BM, HH), lambda n: (n, 0)),
        pl.BlockSpec((1, 2, _BM, 128), lambda n: (0, 0, n, 0)),
        pl.BlockSpec((_BM, HH), lambda n: (n, 0)),
        pl.BlockSpec((1, 2, _BM, 128), lambda n: (0, 0, n, 0)),
    ],
    out_shape=[
        jax.ShapeDtypeStruct((NP, HH), jnp.float32),
        jax.ShapeDtypeStruct((NP, HH), jnp.float32),
        jax.ShapeDtypeStruct((1, 2, NP, 128), jnp.float32),
        jax.ShapeDtypeStruct((NP, HH), jnp.float32),
        jax.ShapeDtypeStruct((1, 2, NP, 128), jnp.float32),
    ],
)

_xprep_call = pl.pallas_call(
    _xprep_body,
    grid=(TT, NP // _BM),
    in_specs=[
        pl.BlockSpec((1, _BM, DD), lambda t, n: (t, n, 0)),
        pl.BlockSpec((_BM, 1), lambda t, n: (n, 0)),
    ],
    out_specs=pl.BlockSpec((1, 2, _BM, 128), lambda t, n: (t, 0, n, 0)),
    out_shape=jax.ShapeDtypeStruct((TT, 2, NP, 128), jnp.float32),
)

_fc_call = pl.pallas_call(
    _fc_body,
    out_shape=jax.ShapeDtypeStruct((NP, 128), jnp.float32),
)


def kernel(x, edge_index, Wx0, Wh0, b0, Wx1, Wh1, b1, Wfc, bfc):
    src = edge_index[0].astype(jnp.int32)
    dst = edge_index[1].astype(jnp.int32)

    pad = EP - EE
    src_p = jnp.concatenate([src, jnp.full((pad,), NN, jnp.int32)])
    dst_p = jnp.concatenate([dst, jnp.full((pad,), NN, jnp.int32)])
    srcs = jnp.stack([src_p, src_p + NP]).reshape(NC, NS, NB, BB)
    dsts = dst_p.reshape(NS, NB, BB)

    zeros_agg = jnp.zeros((ZR, 128), jnp.float32)
    ones_tbl = jnp.ones((2 * NP, 128), jnp.float32)

    sc1 = _sc_agg_multi(1)
    agg = lambda tbl: sc1(tbl, srcs, dsts, zeros_agg)[0].reshape(2, NP, 128)

    (degp,) = sc1(ones_tbl, srcs, dsts, zeros_agg)
    deg = degp[:NP, 0]
    dinv = jax.lax.rsqrt(jnp.clip(deg, 1.0, None)).reshape(NP, 1)

    xpad = jnp.pad(x, ((0, 0), (0, NP - NN), (0, 0)))
    xp = _xprep_call(xpad, dinv).reshape(TT, 2 * NP, 128)

    z2 = jnp.zeros((2, NP, 128), jnp.float32)
    zN = jnp.zeros((NP, HH), jnp.float32)
    b0r = b0.reshape(1, 4 * HH)
    b1r = b1.reshape(1, 4 * HH)

    ax = [agg(xp[t]) for t in range(TT)]

    g0 = z2
    g1 = z2
    c0 = zN
    c1 = zN
    h1 = zN
    for t in range(TT):
        _, c0, hp0 = _cell_call(ax[t], g0, dinv, c0, Wx0, Wh0, b0r)
        g0 = agg(hp0.reshape(2 * NP, 128))
        h1, c1, hp1 = _cell_call(g0, g1, dinv, c1, Wx1, Wh1, b1r)
        if t < TT - 1:
            g1 = agg(hp1.reshape(2 * NP, 128))

    Wfc_pad = jnp.pad(Wfc, ((0, 0), (0, 127)))
    bfc_pad = jnp.pad(bfc, ((0, 127))).reshape(1, 128)
    score = _fc_call(h1, Wfc_pad, bfc_pad)
    return score[:NN, :1]
